# Initial kernel scaffold; baseline (speedup 1.0000x reference)
import jax
import jax.numpy as jnp
from jax.experimental import pallas as pl


def kernel(x, edge_index, W0, b0, Wl1, bl1, Wr1, g1, be1, Wl2, bl2, Wr2, g2, be2, Wl3, bl3, Wr3, g3, be3, Wc, bc):
    raise NotImplementedError("write your pallas kernel here")


# speedup vs baseline: 4.6353x; 4.6353x over previous
import functools

import jax
import jax.numpy as jnp
from jax import lax
from jax.experimental import pallas as pl
from jax.experimental.pallas import tpu as pltpu
from jax.experimental.pallas import tpu_sc as plsc

N = 10000
E = 320000
D = 128
H = 128
C = 40

NC = 2
NS = 16
NW = NC * NS

NP = 10112
NPAD = NP - N
KMAX = 128
EPW = -(-E // (NW * KMAX)) * KMAX
EP = EPW * NW
RPS = NP // NS
DEGW = 16


K = 128
NBLK = EPW // K

_Z_CHUNKS = [(t * K, K) for t in range(RPS // K)]
if RPS % K:
  _Z_CHUNKS.append((RPS - RPS % K, RPS % K))


def _mesh():
  return plsc.VectorSubcoreMesh(
      core_axis_name="c", subcore_axis_name="s", num_cores=NC, num_subcores=NS)


@functools.lru_cache(maxsize=None)
def _make_sc_segsum():
  scratch = [
      pltpu.VMEM((K,), jnp.int32),
      pltpu.VMEM((K,), jnp.int32),
      pltpu.VMEM((K, H), jnp.float32),
      pltpu.VMEM_SHARED((NP, H), jnp.float32),
      pltpu.SemaphoreType.DMA,
  ]

  def body(h_hbm, src_hbm, dst_hbm, out_hbm, sidx, didx, rows, acc, gsem):
    cid = lax.axis_index("c")
    sid = lax.axis_index("s")
    wid = sid * NC + cid

    zero16 = jnp.zeros((16,), jnp.float32)

    def zrow(r, carry):
      for j in range(H // 16):
        rows[r, pl.ds(j * 16, 16)] = zero16
      return carry

    lax.fori_loop(0, K, zrow, 0)
    for off, sz in _Z_CHUNKS:
      pltpu.sync_copy(rows.at[pl.ds(0, sz)],
                      acc.at[pl.ds(sid * RPS + off, sz)])

    plsc.subcore_barrier()

    def eblk(j, carry):
      off = wid * EPW + j * K
      pltpu.sync_copy(src_hbm.at[pl.ds(off, K)], sidx)
      pltpu.sync_copy(dst_hbm.at[pl.ds(off, K)], didx)
      pltpu.async_copy(h_hbm.at[sidx], rows, gsem).wait()
      pltpu.sync_copy(rows, acc.at[didx], add=True)
      return carry

    lax.fori_loop(0, NBLK, eblk, 0)

    plsc.subcore_barrier()

    pltpu.sync_copy(acc.at[pl.ds(sid * RPS, RPS)],
                    out_hbm.at[cid, pl.ds(sid * RPS, RPS)])

  return pl.kernel(
      body, out_type=[jax.ShapeDtypeStruct((NC, NP, H), jnp.float32)],
      mesh=_mesh(), scratch_types=scratch)


def _tc_pre(x, W0, b0):
  def body(x_ref, w_ref, b_ref, o_ref):
    h = jnp.dot(x_ref[...], w_ref[...],
                preferred_element_type=jnp.float32) + b_ref[...]
    o_ref[0:N, :] = h
    o_ref[N:NP, :] = jnp.zeros((NPAD, H), jnp.float32)

  return pl.pallas_call(
      body,
      out_shape=jax.ShapeDtypeStruct((NP, H), jnp.float32),
  )(x, W0, b0.reshape(1, H))


def _layer_compute(p, d, h, wl, bl, wr, g, be):
  deg = jnp.maximum(d[0, :, 0:1] + d[1, :, 0:1], 1.0)
  agg = (p[0] + p[1]) / deg
  z = (jnp.dot(agg, wl, preferred_element_type=jnp.float32) + bl
       + jnp.dot(h, wr, preferred_element_type=jnp.float32))
  zv = z[0:N, :]
  mu = jnp.mean(zv, axis=0, keepdims=True)
  var = jnp.mean((zv - mu) ** 2, axis=0, keepdims=True)
  hn = (z - mu) * lax.rsqrt(var + 1e-5) * g + be
  return jnp.maximum(hn, 0.0)


def _tc_layer(p, dcnt, h, Wl, bl, Wr, g, be):
  def body(p_ref, d_ref, h_ref, wl_ref, bl_ref, wr_ref, g_ref, be_ref, o_ref):
    o_ref[...] = _layer_compute(p_ref[...], d_ref[...], h_ref[...],
                                wl_ref[...], bl_ref[...], wr_ref[...],
                                g_ref[...], be_ref[...])

  return pl.pallas_call(
      body,
      out_shape=jax.ShapeDtypeStruct((NP, H), jnp.float32),
  )(p, dcnt, h, Wl, bl.reshape(1, H), Wr, g.reshape(1, H), be.reshape(1, H))


def _tc_final(p, dcnt, h, Wl, bl, Wr, g, be, Wc, bc):
  def body(p_ref, d_ref, h_ref, wl_ref, bl_ref, wr_ref, g_ref, be_ref,
           wc_ref, bc_ref, o_ref):
    hn = _layer_compute(p_ref[...], d_ref[...], h_ref[...],
                        wl_ref[...], bl_ref[...], wr_ref[...],
                        g_ref[...], be_ref[...])
    logits = jnp.dot(hn[0:N, :], wc_ref[...],
                     preferred_element_type=jnp.float32) + bc_ref[...]
    m = jnp.max(logits, axis=-1, keepdims=True)
    e = jnp.exp(logits - m)
    o_ref[...] = e / jnp.sum(e, axis=-1, keepdims=True)

  return pl.pallas_call(
      body,
      out_shape=jax.ShapeDtypeStruct((N, C), jnp.float32),
  )(p, dcnt, h, Wl, bl.reshape(1, H), Wr, g.reshape(1, H), be.reshape(1, H),
    Wc, bc.reshape(1, C))


def kernel(x, edge_index, W0, b0, Wl1, bl1, Wr1, g1, be1, Wl2, bl2, Wr2, g2,
           be2, Wl3, bl3, Wr3, g3, be3, Wc, bc):
  src = edge_index[0].astype(jnp.int32)
  dst = edge_index[1].astype(jnp.int32)
  pad_ids = (jnp.arange(EP - E, dtype=jnp.int32) % NPAD) + N
  srcp = jnp.concatenate([src, pad_ids])
  dstp = jnp.concatenate([dst, pad_ids])

  h0 = _tc_pre(x, W0, b0)
  ones_tab = jnp.ones((NP, H), jnp.float32)
  (dcnt,) = _make_sc_segsum()(ones_tab, srcp, dstp)
  (p1,) = _make_sc_segsum()(h0, srcp, dstp)
  h1 = _tc_layer(p1, dcnt, h0, Wl1, bl1, Wr1, g1, be1)
  (p2,) = _make_sc_segsum()(h1, srcp, dstp)
  h2 = _tc_layer(p2, dcnt, h1, Wl2, bl2, Wr2, g2, be2)
  (p3,) = _make_sc_segsum()(h2, srcp, dstp)
  return _tc_final(p3, dcnt, h2, Wl3, bl3, Wr3, g3, be3, Wc, bc)


# speedup vs baseline: 5.3354x; 1.1510x over previous
import functools

import jax
import jax.numpy as jnp
from jax import lax
from jax.experimental import pallas as pl
from jax.experimental.pallas import tpu as pltpu
from jax.experimental.pallas import tpu_sc as plsc

N = 10000
E = 320000
D = 128
H = 128
C = 40

NC = 2
NS = 16
NW = NC * NS

NP = 10112
NPAD = NP - N
KMAX = 128
EPW = -(-E // (NW * KMAX)) * KMAX
EP = EPW * NW
RPS = NP // NS
DEGW = 16


K = 128
NBLK = EPW // K

_Z_CHUNKS = [(t * K, K) for t in range(RPS // K)]
if RPS % K:
  _Z_CHUNKS.append((RPS - RPS % K, RPS % K))


def _mesh():
  return plsc.VectorSubcoreMesh(
      core_axis_name="c", subcore_axis_name="s", num_cores=NC, num_subcores=NS)


@functools.lru_cache(maxsize=None)
def _make_sc_segsum():
  scratch = [
      pltpu.VMEM((K,), jnp.int32),
      pltpu.VMEM((K,), jnp.int32),
      pltpu.VMEM((K, H), jnp.float32),
      pltpu.VMEM_SHARED((NP, H), jnp.float32),
      pltpu.SemaphoreType.DMA,
  ]

  def body(h_hbm, src_hbm, dst_hbm, out_hbm, sidx, didx, rows, acc, gsem):
    cid = lax.axis_index("c")
    sid = lax.axis_index("s")
    wid = sid * NC + cid

    zero16 = jnp.zeros((16,), jnp.float32)

    def zrow(r, carry):
      for j in range(H // 16):
        rows[r, pl.ds(j * 16, 16)] = zero16
      return carry

    lax.fori_loop(0, K, zrow, 0)
    for off, sz in _Z_CHUNKS:
      pltpu.sync_copy(rows.at[pl.ds(0, sz)],
                      acc.at[pl.ds(sid * RPS + off, sz)])

    plsc.subcore_barrier()

    def eblk(j, carry):
      off = wid * EPW + j * K
      pltpu.sync_copy(src_hbm.at[pl.ds(off, K)], sidx)
      pltpu.sync_copy(dst_hbm.at[pl.ds(off, K)], didx)
      pltpu.async_copy(h_hbm.at[sidx], rows, gsem).wait()
      pltpu.sync_copy(rows, acc.at[didx], add=True)
      return carry

    lax.fori_loop(0, NBLK, eblk, 0)

    plsc.subcore_barrier()

    pltpu.sync_copy(acc.at[pl.ds(sid * RPS, RPS)],
                    out_hbm.at[cid, pl.ds(sid * RPS, RPS)])

  return pl.kernel(
      body, out_type=[jax.ShapeDtypeStruct((NC, NP, H), jnp.float32)],
      mesh=_mesh(), scratch_types=scratch)


@functools.lru_cache(maxsize=None)
def _make_sc_degree():
  scratch = [
      pltpu.VMEM((K,), jnp.int32),
      pltpu.VMEM((K, H), jnp.float32),
      pltpu.VMEM_SHARED((NP, H), jnp.float32),
  ]

  def body(dst_hbm, out_hbm, didx, rows, acc):
    cid = lax.axis_index("c")
    sid = lax.axis_index("s")
    wid = sid * NC + cid
    zero16 = jnp.zeros((16,), jnp.float32)
    one16 = jnp.ones((16,), jnp.float32)

    def zrow(r, carry):
      for j in range(H // 16):
        rows[r, pl.ds(j * 16, 16)] = zero16
      return carry

    lax.fori_loop(0, K, zrow, 0)
    for off, sz in _Z_CHUNKS:
      pltpu.sync_copy(rows.at[pl.ds(0, sz)],
                      acc.at[pl.ds(sid * RPS + off, sz)])

    def frow(r, carry):
      for j in range(H // 16):
        rows[r, pl.ds(j * 16, 16)] = one16
      return carry

    lax.fori_loop(0, K, frow, 0)
    plsc.subcore_barrier()

    def eblk(j, carry):
      off = wid * EPW + j * K
      pltpu.sync_copy(dst_hbm.at[pl.ds(off, K)], didx)
      pltpu.sync_copy(rows, acc.at[didx], add=True)
      return carry

    lax.fori_loop(0, NBLK, eblk, 0)
    plsc.subcore_barrier()
    pltpu.sync_copy(acc.at[pl.ds(sid * RPS, RPS)],
                    out_hbm.at[cid, pl.ds(sid * RPS, RPS)])

  return pl.kernel(
      body, out_type=[jax.ShapeDtypeStruct((NC, NP, H), jnp.float32)],
      mesh=_mesh(), scratch_types=scratch)


def _tc_pre(x, W0, b0):
  def body(x_ref, w_ref, b_ref, o_ref):
    h = jnp.dot(x_ref[...], w_ref[...],
                preferred_element_type=jnp.float32) + b_ref[...]
    o_ref[0:N, :] = h
    o_ref[N:NP, :] = jnp.zeros((NPAD, H), jnp.float32)

  return pl.pallas_call(
      body,
      out_shape=jax.ShapeDtypeStruct((NP, H), jnp.float32),
  )(x, W0, b0.reshape(1, H))


def _layer_compute(p, d, h, wl, bl, wr, g, be):
  deg = jnp.maximum(d[0, :, 0:1] + d[1, :, 0:1], 1.0)
  agg = (p[0] + p[1]) / deg
  z = (jnp.dot(agg, wl, preferred_element_type=jnp.float32) + bl
       + jnp.dot(h, wr, preferred_element_type=jnp.float32))
  zv = z[0:N, :]
  mu = jnp.mean(zv, axis=0, keepdims=True)
  var = jnp.mean((zv - mu) ** 2, axis=0, keepdims=True)
  hn = (z - mu) * lax.rsqrt(var + 1e-5) * g + be
  return jnp.maximum(hn, 0.0)


def _tc_layer(p, dcnt, h, Wl, bl, Wr, g, be):
  def body(p_ref, d_ref, h_ref, wl_ref, bl_ref, wr_ref, g_ref, be_ref, o_ref):
    o_ref[...] = _layer_compute(p_ref[...], d_ref[...], h_ref[...],
                                wl_ref[...], bl_ref[...], wr_ref[...],
                                g_ref[...], be_ref[...])

  return pl.pallas_call(
      body,
      out_shape=jax.ShapeDtypeStruct((NP, H), jnp.float32),
  )(p, dcnt, h, Wl, bl.reshape(1, H), Wr, g.reshape(1, H), be.reshape(1, H))


def _tc_final(p, dcnt, h, Wl, bl, Wr, g, be, Wc, bc):
  def body(p_ref, d_ref, h_ref, wl_ref, bl_ref, wr_ref, g_ref, be_ref,
           wc_ref, bc_ref, o_ref):
    hn = _layer_compute(p_ref[...], d_ref[...], h_ref[...],
                        wl_ref[...], bl_ref[...], wr_ref[...],
                        g_ref[...], be_ref[...])
    logits = jnp.dot(hn[0:N, :], wc_ref[...],
                     preferred_element_type=jnp.float32) + bc_ref[...]
    m = jnp.max(logits, axis=-1, keepdims=True)
    e = jnp.exp(logits - m)
    o_ref[...] = e / jnp.sum(e, axis=-1, keepdims=True)

  return pl.pallas_call(
      body,
      out_shape=jax.ShapeDtypeStruct((N, C), jnp.float32),
  )(p, dcnt, h, Wl, bl.reshape(1, H), Wr, g.reshape(1, H), be.reshape(1, H),
    Wc, bc.reshape(1, C))


def kernel(x, edge_index, W0, b0, Wl1, bl1, Wr1, g1, be1, Wl2, bl2, Wr2, g2,
           be2, Wl3, bl3, Wr3, g3, be3, Wc, bc):
  src = edge_index[0].astype(jnp.int32)
  dst = edge_index[1].astype(jnp.int32)
  pad_ids = (jnp.arange(EP - E, dtype=jnp.int32) % NPAD) + N
  srcp = jnp.concatenate([src, pad_ids])
  dstp = jnp.concatenate([dst, pad_ids])

  h0 = _tc_pre(x, W0, b0)
  (dcnt,) = _make_sc_degree()(dstp)
  (p1,) = _make_sc_segsum()(h0, srcp, dstp)
  h1 = _tc_layer(p1, dcnt, h0, Wl1, bl1, Wr1, g1, be1)
  (p2,) = _make_sc_segsum()(h1, srcp, dstp)
  h2 = _tc_layer(p2, dcnt, h1, Wl2, bl2, Wr2, g2, be2)
  (p3,) = _make_sc_segsum()(h2, srcp, dstp)
  return _tc_final(p3, dcnt, h2, Wl3, bl3, Wr3, g3, be3, Wc, bc)
